# Initial kernel scaffold; baseline (speedup 1.0000x reference)
import jax
import jax.numpy as jnp
from jax.experimental import pallas as pl


def kernel(x, species_emb, ability_emb, item_emb, move_emb, group_idx):
    raise NotImplementedError("write your pallas kernel here")


# speedup vs baseline: 5.9191x; 5.9191x over previous
import functools
import jax
import jax.numpy as jnp
from jax import lax
from jax.experimental import pallas as pl
from jax.experimental.pallas import tpu as pltpu
from jax.experimental.pallas import tpu_sc as plsc

BATCH, SEQ, GSIZE = 4096, 50, 8
N = BATCH * SEQ
D_SP, D_AB, D_IT, D_MV = 32, 16, 16, 16
D_OUT = D_SP + D_AB + D_IT + D_MV + 4

_info = plsc.get_sparse_core_info()
NC, NS, L = _info.num_cores, _info.num_subcores, _info.num_lanes
NW = NC * NS
PER_W = N // NW
B = 128
CHUNKS = PER_W // B


def _make_kernel():
    mesh = plsc.VectorSubcoreMesh(core_axis_name="c", subcore_axis_name="s")

    @functools.partial(
        pl.kernel,
        mesh=mesh,
        out_type=jax.ShapeDtypeStruct((N * D_OUT,), jnp.float32),
        compiler_params=pltpu.CompilerParams(
            needs_layout_passes=False, use_tc_tiling_on_sc=False),
        scratch_types=[
            pltpu.VMEM((B * GSIZE,), jnp.float32),
            pltpu.VMEM((B,), jnp.int32),
            pltpu.VMEM((B,), jnp.int32),
            pltpu.VMEM((B,), jnp.int32),
            pltpu.VMEM((B,), jnp.int32),
            pltpu.VMEM((B, D_SP), jnp.float32),
            pltpu.VMEM((B, D_AB), jnp.float32),
            pltpu.VMEM((B, D_IT), jnp.float32),
            pltpu.VMEM((B, D_MV), jnp.float32),
            pltpu.VMEM((B * D_OUT,), jnp.float32),
            pltpu.SemaphoreType.DMA,
        ],
    )
    def k(x_hbm, sp_hbm, ab_hbm, it_hbm, mv_hbm, out_hbm,
          x_v, i0, i1, i2, i3, sp_v, ab_v, it_v, mv_v, out_v, sem):
        wid = lax.axis_index("s") * NC + lax.axis_index("c")
        lane = lax.iota(jnp.int32, L)
        lane8 = lane * GSIZE
        rq = lax.shift_right_logical(lane, 2)
        cq = lax.bitwise_and(lane, 3)
        tsrc = rq * GSIZE + cq + 4
        tdst = rq * D_OUT + cq + (D_OUT - 4)

        def chunk(j, carry):
            base = wid * PER_W + j * B
            pltpu.sync_copy(x_hbm.at[pl.ds(base * GSIZE, B * GSIZE)], x_v)

            for kk in range(B // L):
                for col, iv in ((0, i0), (1, i1), (2, i2), (3, i3)):
                    vals = plsc.load_gather(x_v, [lane8 + (kk * L * GSIZE + col)])
                    ids = jnp.maximum(vals.astype(jnp.int32), 0)
                    iv[pl.ds(kk * L, L)] = ids

            c0 = pltpu.async_copy(sp_hbm.at[i0], sp_v, sem)
            c1 = pltpu.async_copy(ab_hbm.at[i1], ab_v, sem)
            c2 = pltpu.async_copy(it_hbm.at[i2], it_v, sem)
            c3 = pltpu.async_copy(mv_hbm.at[i3], mv_v, sem)
            c0.wait(); c1.wait(); c2.wait(); c3.wait()

            for q in range(B // 4):
                for rr in range(4):
                    r = 4 * q + rr
                    o = r * D_OUT
                    out_v[pl.ds(o, L)] = sp_v[r, pl.ds(0, L)]
                    out_v[pl.ds(o + L, L)] = sp_v[r, pl.ds(L, L)]
                    out_v[pl.ds(o + 32, L)] = ab_v[r, :]
                    out_v[pl.ds(o + 48, L)] = it_v[r, :]
                    out_v[pl.ds(o + 64, L)] = mv_v[r, :]
                tail = plsc.load_gather(x_v, [tsrc + 4 * q * GSIZE])
                plsc.store_scatter(out_v, [tdst + 4 * q * D_OUT], tail)

            pltpu.sync_copy(out_v, out_hbm.at[pl.ds(base * D_OUT, B * D_OUT)])
            return carry

        lax.fori_loop(0, CHUNKS, chunk, 0)

    return k


_sc_lookup = _make_kernel()


def kernel(x, species_emb, ability_emb, item_emb, move_emb, group_idx):
    x1 = x.reshape(N * GSIZE)
    out = _sc_lookup(x1, species_emb, ability_emb, item_emb, move_emb)
    return out.reshape(BATCH, SEQ, D_OUT)
